# baseline (speedup 1.0000x reference)
import functools

import jax
import jax.numpy as jnp
from jax import lax
from jax.experimental import pallas as pl
from jax.experimental.pallas import tpu as pltpu
from jax.experimental.pallas import tpu_sc as plsc

_L = 16
_BW = 512
_GRP = 128


def kernel(x, weight, weight_scale):
    B, H = x.shape
    V, D = weight.shape
    HP = (H + 7) // 8 * 8

    info = plsc.get_sparse_core_info()
    NC, NS = info.num_cores, info.num_subcores
    NW = NC * NS
    assert B == NW * _BW and V % 4 == 0 and D == 32

    w4 = weight.reshape(V // 4, 4 * D)
    xT = x.T

    mesh = plsc.VectorSubcoreMesh(core_axis_name="c", subcore_axis_name="s")

    @functools.partial(
        pl.kernel,
        mesh=mesh,
        compiler_params=pltpu.CompilerParams(use_tc_tiling_on_sc=True, needs_layout_passes=False),
        out_type=jax.ShapeDtypeStruct((H, D, B), jnp.float32),
        scratch_types=[
            pltpu.VMEM((HP, _BW), jnp.int32),
            pltpu.VMEM((_BW,), jnp.int32),
            pltpu.VMEM((_BW, 4 * D), jnp.float32),
            pltpu.VMEM((D, _BW), jnp.float32),
            pltpu.SemaphoreType.DMA,
        ],
    )
    def gather_k(w4_hbm, xt_hbm, out_hbm, idx_t, idx4, rows, plane, sem):
        wid = lax.axis_index("s") * NC + lax.axis_index("c")
        b0 = pl.multiple_of(wid * _BW, _BW)

        for hb in range(H // 8):
            for bb in range(_BW // 128):
                pltpu.sync_copy(
                    xt_hbm.at[pl.ds(hb * 8, 8), pl.ds(b0 + bb * 128, 128)],
                    idx_t.at[pl.ds(hb * 8, 8), pl.ds(bb * 128, 128)])
        if H % 8:
            for bb in range(_BW // 128):
                pltpu.sync_copy(
                    xt_hbm.at[pl.ds(H // 8 * 8, H % 8),
                              pl.ds(b0 + bb * 128, 128)],
                    idx_t.at[pl.ds(H // 8 * 8, H % 8), pl.ds(bb * 128, 128)])

        lane = lax.iota(jnp.int32, _L)

        def h_body(h, carry):
            for g in range(_BW // _L):
                v = idx_t[h, pl.ds(g * _L, _L)]
                idx4[pl.ds(g * _L, _L)] = lax.shift_right_logical(v, 2)
            copies = [
                pltpu.async_copy(
                    w4_hbm.at[idx4.at[pl.ds(c * _GRP, _GRP)]],
                    rows.at[pl.ds(c * _GRP, _GRP)],
                    sem,
                )
                for c in range(_BW // _GRP)
            ]
            for cp in copies:
                cp.wait()
            for g in range(_BW // _L):
                v = idx_t[h, pl.ds(g * _L, _L)]
                word0 = lax.shift_left(jnp.bitwise_and(v, 3), 5)
                row_id = lane + g * _L
                for e in range(D):
                    plane[e, pl.ds(g * _L, _L)] = plsc.load_gather(
                        rows, [row_id, word0 + e])
            for eb in range(D // 8):
                for bb in range(_BW // 128):
                    pltpu.sync_copy(
                        plane.at[pl.ds(eb * 8, 8), pl.ds(bb * 128, 128)],
                        out_hbm.at[h, pl.ds(eb * 8, 8),
                                   pl.ds(b0 + bb * 128, 128)])
            return carry

        lax.fori_loop(0, H, h_body, 0)

    out_t = gather_k(w4, xT)
    return jnp.transpose(out_t, (2, 0, 1))


# speedup vs baseline: 1.1784x; 1.1784x over previous
import functools

import jax
import jax.numpy as jnp
from jax import lax
from jax.experimental import pallas as pl
from jax.experimental.pallas import tpu as pltpu
from jax.experimental.pallas import tpu_sc as plsc

_L = 16
_BW = 512
_HALF = 256
_GRP = 128


def kernel(x, weight, weight_scale):
    B, H = x.shape
    V, D = weight.shape
    HP = (H + 7) // 8 * 8

    info = plsc.get_sparse_core_info()
    NC, NS = info.num_cores, info.num_subcores
    NW = NC * NS
    assert B == NW * _BW and V % 4 == 0 and D == 32

    w4 = weight.reshape(V // 4, 4 * D)
    xT = x.T
    NV = 2 * H

    mesh = plsc.VectorSubcoreMesh(core_axis_name="c", subcore_axis_name="s")

    @functools.partial(
        pl.kernel,
        mesh=mesh,
        compiler_params=pltpu.CompilerParams(
            use_tc_tiling_on_sc=True, needs_layout_passes=False),
        out_type=jax.ShapeDtypeStruct((H, D, B), jnp.float32),
        scratch_types=[
            pltpu.VMEM((HP, _BW), jnp.int32),
            pltpu.VMEM((_HALF,), jnp.int32),
            pltpu.VMEM((_HALF,), jnp.int32),
            pltpu.VMEM((_HALF, 4 * D), jnp.float32),
            pltpu.VMEM((_HALF, 4 * D), jnp.float32),
            pltpu.VMEM((D, _BW), jnp.float32),
            pltpu.SemaphoreType.DMA,
            pltpu.SemaphoreType.DMA,
            pltpu.SemaphoreType.DMA,
        ],
    )
    def gather_k(w4_hbm, xt_hbm, out_hbm, idx_t, i40, i41, r0, r1, plane,
                 g0, g1, ssem):
        idx4 = (i40, i41)
        rows = (r0, r1)
        gsem = (g0, g1)
        wid = lax.axis_index("s") * NC + lax.axis_index("c")
        b0 = pl.multiple_of(wid * _BW, _BW)

        for hb in range(H // 8):
            for bb in range(_BW // 128):
                pltpu.sync_copy(
                    xt_hbm.at[pl.ds(hb * 8, 8), pl.ds(b0 + bb * 128, 128)],
                    idx_t.at[pl.ds(hb * 8, 8), pl.ds(bb * 128, 128)])
        if H % 8:
            for bb in range(_BW // 128):
                pltpu.sync_copy(
                    xt_hbm.at[pl.ds(H // 8 * 8, H % 8),
                              pl.ds(b0 + bb * 128, 128)],
                    idx_t.at[pl.ds(H // 8 * 8, H % 8), pl.ds(bb * 128, 128)])

        lane = lax.iota(jnp.int32, _L)

        def build_fire(v, q):
            hv = v // 2
            cv = pl.multiple_of((v % 2) * _HALF, _HALF)
            for g in range(_HALF // _L):
                vv = idx_t[hv, pl.ds(cv + g * _L, _L)]
                idx4[q][pl.ds(g * _L, _L)] = lax.shift_right_logical(vv, 2)
            for s in range(_HALF // _GRP):
                pltpu.async_copy(
                    w4_hbm.at[idx4[q].at[pl.ds(s * _GRP, _GRP)]],
                    rows[q].at[pl.ds(s * _GRP, _GRP)],
                    gsem[q])

        def drain_gather(q):
            pltpu.make_async_copy(
                w4_hbm.at[pl.ds(0, _HALF)], rows[q], gsem[q]).wait()

        def drain_stores():
            pltpu.make_async_copy(
                out_hbm.at[0, pl.ds(0, D), pl.ds(0, _BW)], plane, ssem).wait()

        def extract(v, q):
            hv = v // 2
            cv = pl.multiple_of((v % 2) * _HALF, _HALF)

            def g_body(g, carry):
                o = pl.multiple_of(g * _L, _L)
                vv = idx_t[hv, pl.ds(cv + o, _L)]
                word0 = lax.shift_left(jnp.bitwise_and(vv, 3), 5)
                row_id = lane + o
                for e in range(D):
                    plane[e, pl.ds(cv + o, _L)] = plsc.load_gather(
                        rows[q], [row_id, word0 + e])
                return carry

            lax.fori_loop(0, _HALF // _L, g_body, 0)

        def fire_stores(v):
            hv = v // 2
            for eb in range(D // 8):
                for bb in range(_BW // 128):
                    pltpu.async_copy(
                        plane.at[pl.ds(eb * 8, 8), pl.ds(bb * 128, 128)],
                        out_hbm.at[hv, pl.ds(eb * 8, 8),
                                   pl.ds(b0 + bb * 128, 128)],
                        ssem)

        def visit(v, q, drain_st, fire_next, last_h):
            if fire_next:
                build_fire(v + 1, 1 - q)
            drain_gather(q)
            if drain_st:
                drain_stores()
            extract(v, q)
            if last_h:
                fire_stores(v)

        build_fire(0, 0)
        visit(0, 0, drain_st=False, fire_next=True, last_h=False)
        visit(1, 1, drain_st=False, fire_next=True, last_h=True)

        def steady(t, carry):
            v = 2 + 2 * t
            visit(v, 0, drain_st=True, fire_next=True, last_h=False)
            visit(v + 1, 1, drain_st=False, fire_next=True, last_h=True)
            return carry

        lax.fori_loop(0, (NV - 4) // 2, steady, 0)
        visit(NV - 2, 0, drain_st=True, fire_next=True, last_h=False)
        visit(NV - 1, 1, drain_st=False, fire_next=False, last_h=True)
        drain_stores()

    out_t = gather_k(w4, xT)
    return jnp.transpose(out_t, (2, 0, 1))


# speedup vs baseline: 1.3626x; 1.1563x over previous
import functools

import jax
import jax.numpy as jnp
from jax import lax
from jax.experimental import pallas as pl
from jax.experimental.pallas import tpu as pltpu
from jax.experimental.pallas import tpu_sc as plsc

_NX = 16


def kernel(x, weight, weight_scale):
    B, H = x.shape
    V, D = weight.shape

    info = plsc.get_sparse_core_info()
    NC, NS = info.num_cores, info.num_subcores
    NW = NC * NS
    assert B % (NW * _NX) == 0
    xrows_per_w = B // NW
    chunks = xrows_per_w // _NX
    assert chunks % 2 == 0 and chunks >= 4

    mesh = plsc.VectorSubcoreMesh(core_axis_name="c", subcore_axis_name="s")

    @functools.partial(
        pl.kernel,
        mesh=mesh,
        compiler_params=pltpu.CompilerParams(use_tc_tiling_on_sc=False),
        out_type=jax.ShapeDtypeStruct((B, H, D), jnp.float32),
        scratch_types=[
            pltpu.VMEM((xrows_per_w, H), jnp.int32),
            pltpu.VMEM((_NX, H, D), jnp.float32),
            pltpu.VMEM((_NX, H, D), jnp.float32),
            pltpu.SemaphoreType.DMA,
            pltpu.SemaphoreType.DMA,
            pltpu.SemaphoreType.DMA,
            pltpu.SemaphoreType.DMA,
        ],
    )
    def gather_k(table_hbm, idx_hbm, out_hbm, idx_all, rows0, rows1,
                 g0, g1, s0, s1):
        rows = (rows0, rows1)
        gsem = (g0, g1)
        ssem = (s0, s1)
        wid = lax.axis_index("s") * NC + lax.axis_index("c")
        row_base = wid * xrows_per_w

        pltpu.sync_copy(idx_hbm.at[pl.ds(row_base, xrows_per_w)], idx_all)

        def fire_gather(c, b):
            for j in range(_NX):
                pltpu.async_copy(
                    table_hbm.at[idx_all.at[c * _NX + j]],
                    rows[b].at[j],
                    gsem[b],
                )

        def drain_gather(b):
            pltpu.make_async_copy(
                out_hbm.at[pl.ds(0, _NX)], rows[b], gsem[b]).wait()

        def fire_store(c, b):
            pltpu.async_copy(
                rows[b], out_hbm.at[pl.ds(row_base + c * _NX, _NX)], ssem[b])

        def drain_store(b):
            pltpu.make_async_copy(
                out_hbm.at[pl.ds(0, _NX)], rows[b], ssem[b]).wait()

        def visit(c, b, drain_nb, fire_next):
            nb = 1 - b
            if fire_next:
                if drain_nb:
                    drain_store(nb)
                fire_gather(c + 1, nb)
            drain_gather(b)
            fire_store(c, b)

        fire_gather(0, 0)
        visit(0, 0, drain_nb=False, fire_next=True)

        def steady(t, carry):
            visit(1 + 2 * t, 1, drain_nb=True, fire_next=True)
            visit(2 + 2 * t, 0, drain_nb=True, fire_next=True)
            return carry

        lax.fori_loop(0, (chunks - 2) // 2, steady, 0)
        visit(chunks - 1, 1, drain_nb=True, fire_next=False)
        drain_store(0)
        drain_store(1)

    return gather_k(weight, x)
